# baseline (speedup 1.0000x reference)
import functools

import jax
import jax.numpy as jnp
from jax import lax
from jax.experimental import pallas as pl
from jax.experimental.pallas import tpu as pltpu
from jax.experimental.pallas import tpu_sc as plsc

N_NODES = 10000
D = 256
DQ = 64
EPS = 1e-5
N_EDGES = 160000

NT = 16
EPT = N_EDGES // NT
CHUNK = 64
NCHUNK = 160
NCHUNK_ALLOC = NCHUNK + 4
EPT_PAD = NCHUNK_ALLOC * CHUNK
N_ACC = 10240
ROWS_PER_TILE = N_ACC // NT
TSTRIPE = 624
TLAST = N_NODES - 15 * TSTRIPE


def _sc_segment_sum(src_t, dst_t, x, zeros_q, ones_c):
    mesh = plsc.VectorSubcoreMesh(core_axis_name="c", subcore_axis_name="s")

    @functools.partial(
        pl.kernel,
        mesh=mesh,
        compiler_params=pltpu.CompilerParams(use_tc_tiling_on_sc=False),
        out_type=(
            jax.ShapeDtypeStruct((4, N_ACC, DQ), jnp.float32),
            jax.ShapeDtypeStruct((2, N_ACC), jnp.float32),
        ),
        scratch_types=[
            pltpu.VMEM((NCHUNK_ALLOC, CHUNK), jnp.int32),
            pltpu.VMEM((NCHUNK_ALLOC, CHUNK), jnp.int32),
            pltpu.VMEM((CHUNK, DQ), jnp.float32),
            pltpu.VMEM((CHUNK, DQ), jnp.float32),
            pltpu.VMEM((CHUNK, DQ), jnp.float32),
            pltpu.VMEM((CHUNK, DQ), jnp.float32),
            pltpu.VMEM((CHUNK,), jnp.float32),
            pltpu.VMEM_SHARED((N_NODES, DQ), jnp.float32),
            pltpu.VMEM_SHARED((N_ACC, DQ), jnp.float32),
            pltpu.VMEM_SHARED((N_ACC,), jnp.float32),
            pltpu.SemaphoreType.DMA,
            pltpu.SemaphoreType.DMA,
            pltpu.SemaphoreType.DMA,
            pltpu.SemaphoreType.DMA,
            pltpu.SemaphoreType.DMA,
            pltpu.SemaphoreType.DMA,
            pltpu.SemaphoreType.DMA,
            pltpu.SemaphoreType.DMA,
        ],
    )
    def seg_sum(src_hbm, dst_hbm, x_hbm, z_hbm, o_hbm,
                sum_hbm, cnt_hbm,
                src_v, dst_v, rows0, rows1, rows2, rows3, ones_v,
                tab_sh, acc_sh, cnt_sh,
                semg0, semg1, semg2, semg3, sems0, sems1, sems2, sems3):
        c = lax.axis_index("c")
        s = lax.axis_index("s")

        pltpu.sync_copy(z_hbm, rows0)
        pltpu.sync_copy(o_hbm, ones_v)
        pltpu.sync_copy(src_hbm.at[s], src_v)
        pltpu.sync_copy(dst_hbm.at[s], dst_v)

        base = s * ROWS_PER_TILE

        for q in range(2):
            qidx = 2 * c + q
            qcol = qidx * DQ

            @pl.when(s < 15)
            def _():
                pltpu.sync_copy(
                    x_hbm.at[pl.ds(s * TSTRIPE, TSTRIPE), pl.ds(qcol, DQ)],
                    tab_sh.at[pl.ds(s * TSTRIPE, TSTRIPE)])

            @pl.when(s == 15)
            def _():
                pltpu.sync_copy(
                    x_hbm.at[pl.ds(15 * TSTRIPE, TLAST), pl.ds(qcol, DQ)],
                    tab_sh.at[pl.ds(15 * TSTRIPE, TLAST)])
            for k in range(ROWS_PER_TILE // CHUNK):
                pltpu.sync_copy(rows0, acc_sh.at[pl.ds(base + k * CHUNK, CHUNK)])
            if q == 0:
                for k in range(ROWS_PER_TILE // DQ):
                    pltpu.sync_copy(rows0.at[0],
                                    cnt_sh.at[pl.ds(base + k * DQ, DQ)])
            plsc.subcore_barrier()

            bufs = (rows0, rows1, rows2, rows3)
            semg = (semg0, semg1, semg2, semg3)
            sems = (sems0, sems1, sems2, sems3)

            def edge_loop(cnt_par):
                def step(j, b, warm, jpar):
                    pltpu.make_async_copy(
                        tab_sh.at[src_v.at[j]], bufs[b], semg[b]).wait()
                    if cnt_par is not None and jpar == cnt_par:
                        pltpu.sync_copy(ones_v, cnt_sh.at[dst_v.at[j]], add=True)
                    pltpu.async_copy(bufs[b], acc_sh.at[dst_v.at[j]], sems[b],
                                     add=True)
                    b2 = (b + 2) % 4
                    if warm:
                        pltpu.make_async_copy(
                            bufs[b2], acc_sh.at[dst_v.at[j - 2]], sems[b2]).wait()
                    pltpu.async_copy(tab_sh.at[src_v.at[j + 2]], bufs[b2],
                                     semg[b2])

                pltpu.async_copy(tab_sh.at[src_v.at[0]], rows0, semg0)
                pltpu.async_copy(tab_sh.at[src_v.at[1]], rows1, semg1)
                step(0, 0, False, 0)
                step(1, 1, False, 1)

                def quad(i, carry):
                    j0 = 4 * i + 2
                    step(j0, 2, True, 0)
                    step(j0 + 1, 3, True, 1)
                    step(j0 + 2, 0, True, 0)
                    step(j0 + 3, 1, True, 1)
                    return carry

                lax.fori_loop(0, (NCHUNK_ALLOC - 4) // 4, quad, 0)
                nl = NCHUNK_ALLOC
                pltpu.make_async_copy(
                    rows0, acc_sh.at[dst_v.at[nl - 4]], sems0).wait()
                pltpu.make_async_copy(
                    rows1, acc_sh.at[dst_v.at[nl - 3]], sems1).wait()
                pltpu.make_async_copy(
                    tab_sh.at[src_v.at[nl - 2]], rows2, semg2).wait()
                pltpu.make_async_copy(
                    tab_sh.at[src_v.at[nl - 1]], rows3, semg3).wait()

            if q == 0:
                @pl.when(c == 0)
                def _():
                    edge_loop(0)

                @pl.when(c == 1)
                def _():
                    edge_loop(1)
            else:
                edge_loop(None)

            plsc.subcore_barrier()

            pltpu.sync_copy(
                acc_sh.at[pl.ds(base, ROWS_PER_TILE)],
                sum_hbm.at[qidx, pl.ds(base, ROWS_PER_TILE)])
            if q == 0:
                pltpu.sync_copy(cnt_sh.at[pl.ds(base, ROWS_PER_TILE)],
                                cnt_hbm.at[c, pl.ds(base, ROWS_PER_TILE)])
            if q == 0:
                pltpu.sync_copy(z_hbm, rows0)

    return seg_sum(src_t, dst_t, x, zeros_q, ones_c)


BR = 1000
NB = N_NODES // BR


def _tc_fused_body(sum_ref, cnt_ref, x_ref, wl_ref, bl_ref, wr_ref,
                   gamma_ref, beta_ref, out_ref, conv_scr, acc_ref):
    p = pl.program_id(0)
    i = pl.program_id(1)

    @pl.when(p == 0)
    def _():
        recip = 1.0 / jnp.maximum(cnt_ref[0] + cnt_ref[1], 1.0)
        dn = (((1,), (1,)), ((), ()))
        wl = wl_ref[...]
        conv = lax.dot_general(x_ref[...], wr_ref[...], dn,
                               preferred_element_type=jnp.float32)
        for qq in range(4):
            mean_q = sum_ref[qq] * recip
            conv += lax.dot_general(mean_q, wl[:, qq * DQ:(qq + 1) * DQ], dn,
                                    preferred_element_type=jnp.float32)
        conv += bl_ref[...]
        conv_scr[pl.ds(i * BR, BR), :] = conv

        @pl.when(i == 0)
        def _():
            acc_ref[...] = jnp.zeros_like(acc_ref)

        acc_ref[0:1, :] += jnp.sum(conv, axis=0, keepdims=True)
        acc_ref[1:2, :] += jnp.sum(conv * conv, axis=0, keepdims=True)

    @pl.when(p == 1)
    def _():
        n = jnp.float32(N_NODES)
        mu = acc_ref[0:1, :] / n
        var = acc_ref[1:2, :] / n - mu * mu
        inv = lax.rsqrt(var + EPS)
        conv = conv_scr[pl.ds(i * BR, BR), :]
        bn = gamma_ref[...] * (conv - mu) * inv + beta_ref[...]
        out_ref[...] = jnp.maximum(bn, 0.0) + x_ref[...]


def kernel(x, ei, Wl, bl, Wr, gamma, beta):
    src = ei[0].astype(jnp.int32)
    dst = ei[1].astype(jnp.int32)

    pad = EPT_PAD - EPT
    src_t = jnp.concatenate(
        [src.reshape(NT, EPT), jnp.zeros((NT, pad), jnp.int32)], axis=1
    ).reshape(NT, NCHUNK_ALLOC, CHUNK)
    dst_t = jnp.concatenate(
        [dst.reshape(NT, EPT), jnp.full((NT, pad), N_NODES, jnp.int32)], axis=1
    ).reshape(NT, NCHUNK_ALLOC, CHUNK)

    zeros_q = jnp.zeros((CHUNK, DQ), jnp.float32)
    ones_c = jnp.ones((CHUNK,), jnp.float32)

    summed, cnt = _sc_segment_sum(src_t, dst_t, x, zeros_q, ones_c)
    cnt2 = cnt.reshape(2, N_ACC, 1)

    out = pl.pallas_call(
        _tc_fused_body,
        grid=(2, NB),
        in_specs=[
            pl.BlockSpec((4, BR, DQ), lambda p, i: (0, (1 - p) * i, 0)),
            pl.BlockSpec((2, BR, 1), lambda p, i: (0, (1 - p) * i, 0)),
            pl.BlockSpec((BR, D), lambda p, i: (i, 0)),
            pl.BlockSpec((D, D), lambda p, i: (0, 0)),
            pl.BlockSpec((1, D), lambda p, i: (0, 0)),
            pl.BlockSpec((D, D), lambda p, i: (0, 0)),
            pl.BlockSpec((1, D), lambda p, i: (0, 0)),
            pl.BlockSpec((1, D), lambda p, i: (0, 0)),
        ],
        out_specs=pl.BlockSpec((BR, D), lambda p, i: (i, 0)),
        out_shape=jax.ShapeDtypeStruct((N_NODES, D), jnp.float32),
        scratch_shapes=[
            pltpu.VMEM((N_NODES, D), jnp.float32),
            pltpu.VMEM((8, D), jnp.float32),
        ],
    )(summed, cnt2, x, Wl, bl.reshape(1, D), Wr,
      gamma.reshape(1, D), beta.reshape(1, D))

    return out


# speedup vs baseline: 1.0320x; 1.0320x over previous
import functools

import jax
import jax.numpy as jnp
from jax import lax
from jax.experimental import pallas as pl
from jax.experimental.pallas import tpu as pltpu
from jax.experimental.pallas import tpu_sc as plsc

N_NODES = 10000
D = 256
DQ = 64
EPS = 1e-5
N_EDGES = 160000

NT = 16
EPT = N_EDGES // NT
CHUNK = 64
NCHUNK = 160
NCHUNK_ALLOC = NCHUNK + 4
EPT_PAD = NCHUNK_ALLOC * CHUNK
N_ACC = 10240
ROWS_PER_TILE = N_ACC // NT
TSTRIPE = 624
TLAST = N_NODES - 15 * TSTRIPE


def _sc_segment_sum(src_t, dst_t, x, zeros_q, ones_c):
    mesh = plsc.VectorSubcoreMesh(core_axis_name="c", subcore_axis_name="s")

    @functools.partial(
        pl.kernel,
        mesh=mesh,
        compiler_params=pltpu.CompilerParams(use_tc_tiling_on_sc=False),
        out_type=(
            jax.ShapeDtypeStruct((4, N_ACC, DQ), jnp.float32),
            jax.ShapeDtypeStruct((N_ACC,), jnp.float32),
        ),
        scratch_types=[
            pltpu.VMEM((NCHUNK_ALLOC, CHUNK), jnp.int32),
            pltpu.VMEM((NCHUNK_ALLOC, CHUNK), jnp.int32),
            pltpu.VMEM((CHUNK, DQ), jnp.float32),
            pltpu.VMEM((CHUNK, DQ), jnp.float32),
            pltpu.VMEM((CHUNK, DQ), jnp.float32),
            pltpu.VMEM((CHUNK, DQ), jnp.float32),
            pltpu.VMEM((CHUNK,), jnp.float32),
            pltpu.VMEM_SHARED((N_NODES, DQ), jnp.float32),
            pltpu.VMEM_SHARED((N_ACC, DQ), jnp.float32),
            pltpu.VMEM_SHARED((N_ACC,), jnp.float32),
            pltpu.SemaphoreType.DMA,
            pltpu.SemaphoreType.DMA,
            pltpu.SemaphoreType.DMA,
            pltpu.SemaphoreType.DMA,
            pltpu.SemaphoreType.DMA,
            pltpu.SemaphoreType.DMA,
            pltpu.SemaphoreType.DMA,
            pltpu.SemaphoreType.DMA,
            pltpu.SemaphoreType.DMA,
        ],
    )
    def seg_sum(src_hbm, dst_hbm, x_hbm, z_hbm, o_hbm,
                sum_hbm, cnt_hbm,
                src_v, dst_v, rows0, rows1, rows2, rows3, ones_v,
                tab_sh, acc_sh, cnt_sh,
                semg0, semg1, semg2, semg3, sems0, sems1, sems2, sems3, semc):
        c = lax.axis_index("c")
        s = lax.axis_index("s")

        pltpu.sync_copy(z_hbm, rows0)
        pltpu.sync_copy(o_hbm, ones_v)
        pltpu.sync_copy(src_hbm.at[s], src_v)
        pltpu.sync_copy(dst_hbm.at[s], dst_v)

        base = s * ROWS_PER_TILE

        for q in range(2):
            qidx = 2 * c + q
            qcol = qidx * DQ

            @pl.when(s < 15)
            def _():
                pltpu.sync_copy(
                    x_hbm.at[pl.ds(s * TSTRIPE, TSTRIPE), pl.ds(qcol, DQ)],
                    tab_sh.at[pl.ds(s * TSTRIPE, TSTRIPE)])

            @pl.when(s == 15)
            def _():
                pltpu.sync_copy(
                    x_hbm.at[pl.ds(15 * TSTRIPE, TLAST), pl.ds(qcol, DQ)],
                    tab_sh.at[pl.ds(15 * TSTRIPE, TLAST)])
            for k in range(ROWS_PER_TILE // CHUNK):
                pltpu.sync_copy(rows0, acc_sh.at[pl.ds(base + k * CHUNK, CHUNK)])
            if q == 0:
                for k in range(ROWS_PER_TILE // DQ):
                    pltpu.sync_copy(rows0.at[0],
                                    cnt_sh.at[pl.ds(base + k * DQ, DQ)])
            plsc.subcore_barrier()

            bufs = (rows0, rows1, rows2, rows3)
            semg = (semg0, semg1, semg2, semg3)
            sems = (sems0, sems1, sems2, sems3)

            def edge_loop(with_cnt):
                def step(j, b, warm):
                    pltpu.make_async_copy(
                        tab_sh.at[src_v.at[j]], bufs[b], semg[b]).wait()
                    if with_cnt:
                        pltpu.async_copy(ones_v, cnt_sh.at[dst_v.at[j]], semc,
                                         add=True)
                        if warm:
                            pltpu.make_async_copy(
                                ones_v, cnt_sh.at[dst_v.at[j - 2]], semc).wait()
                    pltpu.async_copy(bufs[b], acc_sh.at[dst_v.at[j]], sems[b],
                                     add=True)
                    b2 = (b + 2) % 4
                    if warm:
                        pltpu.make_async_copy(
                            bufs[b2], acc_sh.at[dst_v.at[j - 2]], sems[b2]).wait()
                    pltpu.async_copy(tab_sh.at[src_v.at[j + 2]], bufs[b2],
                                     semg[b2])

                pltpu.async_copy(tab_sh.at[src_v.at[0]], rows0, semg0)
                pltpu.async_copy(tab_sh.at[src_v.at[1]], rows1, semg1)
                step(0, 0, False)
                step(1, 1, False)

                def quad(i, carry):
                    j0 = 4 * i + 2
                    step(j0, 2, True)
                    step(j0 + 1, 3, True)
                    step(j0 + 2, 0, True)
                    step(j0 + 3, 1, True)
                    return carry

                lax.fori_loop(0, (NCHUNK_ALLOC - 4) // 4, quad, 0)
                nl = NCHUNK_ALLOC
                if with_cnt:
                    pltpu.make_async_copy(
                        ones_v, cnt_sh.at[dst_v.at[nl - 4]], semc).wait()
                    pltpu.make_async_copy(
                        ones_v, cnt_sh.at[dst_v.at[nl - 3]], semc).wait()
                pltpu.make_async_copy(
                    rows0, acc_sh.at[dst_v.at[nl - 4]], sems0).wait()
                pltpu.make_async_copy(
                    rows1, acc_sh.at[dst_v.at[nl - 3]], sems1).wait()
                pltpu.make_async_copy(
                    tab_sh.at[src_v.at[nl - 2]], rows2, semg2).wait()
                pltpu.make_async_copy(
                    tab_sh.at[src_v.at[nl - 1]], rows3, semg3).wait()

            if q == 0:
                @pl.when(c == 0)
                def _():
                    edge_loop(True)

                @pl.when(c == 1)
                def _():
                    edge_loop(False)
            else:
                edge_loop(False)

            plsc.subcore_barrier()

            pltpu.sync_copy(
                acc_sh.at[pl.ds(base, ROWS_PER_TILE)],
                sum_hbm.at[qidx, pl.ds(base, ROWS_PER_TILE)])
            if q == 0:
                @pl.when(c == 0)
                def _():
                    pltpu.sync_copy(cnt_sh.at[pl.ds(base, ROWS_PER_TILE)],
                                    cnt_hbm.at[pl.ds(base, ROWS_PER_TILE)])
            if q == 0:
                pltpu.sync_copy(z_hbm, rows0)

    return seg_sum(src_t, dst_t, x, zeros_q, ones_c)


BR = 1000
NB = N_NODES // BR


def _tc_fused_body(sum_ref, cnt_ref, x_ref, wl_ref, bl_ref, wr_ref,
                   gamma_ref, beta_ref, out_ref, conv_scr, acc_ref):
    p = pl.program_id(0)
    i = pl.program_id(1)

    @pl.when(p == 0)
    def _():
        recip = 1.0 / jnp.maximum(cnt_ref[...], 1.0)
        dn = (((1,), (1,)), ((), ()))
        wl = wl_ref[...]
        conv = lax.dot_general(x_ref[...], wr_ref[...], dn,
                               preferred_element_type=jnp.float32)
        for qq in range(4):
            mean_q = sum_ref[qq] * recip
            conv += lax.dot_general(mean_q, wl[:, qq * DQ:(qq + 1) * DQ], dn,
                                    preferred_element_type=jnp.float32)
        conv += bl_ref[...]
        conv_scr[pl.ds(i * BR, BR), :] = conv

        @pl.when(i == 0)
        def _():
            acc_ref[...] = jnp.zeros_like(acc_ref)

        acc_ref[0:1, :] += jnp.sum(conv, axis=0, keepdims=True)
        acc_ref[1:2, :] += jnp.sum(conv * conv, axis=0, keepdims=True)

    @pl.when(p == 1)
    def _():
        n = jnp.float32(N_NODES)
        mu = acc_ref[0:1, :] / n
        var = acc_ref[1:2, :] / n - mu * mu
        inv = lax.rsqrt(var + EPS)
        conv = conv_scr[pl.ds(i * BR, BR), :]
        bn = gamma_ref[...] * (conv - mu) * inv + beta_ref[...]
        out_ref[...] = jnp.maximum(bn, 0.0) + x_ref[...]


def kernel(x, ei, Wl, bl, Wr, gamma, beta):
    src = ei[0].astype(jnp.int32)
    dst = ei[1].astype(jnp.int32)

    pad = EPT_PAD - EPT
    src_t = jnp.concatenate(
        [src.reshape(NT, EPT), jnp.zeros((NT, pad), jnp.int32)], axis=1
    ).reshape(NT, NCHUNK_ALLOC, CHUNK)
    dst_t = jnp.concatenate(
        [dst.reshape(NT, EPT), jnp.full((NT, pad), N_NODES, jnp.int32)], axis=1
    ).reshape(NT, NCHUNK_ALLOC, CHUNK)

    zeros_q = jnp.zeros((CHUNK, DQ), jnp.float32)
    ones_c = jnp.ones((CHUNK,), jnp.float32)

    summed, cnt = _sc_segment_sum(src_t, dst_t, x, zeros_q, ones_c)
    cnt2 = cnt.reshape(N_ACC, 1)

    out = pl.pallas_call(
        _tc_fused_body,
        grid=(2, NB),
        in_specs=[
            pl.BlockSpec((4, BR, DQ), lambda p, i: (0, (1 - p) * i, 0)),
            pl.BlockSpec((BR, 1), lambda p, i: ((1 - p) * i, 0)),
            pl.BlockSpec((BR, D), lambda p, i: (i, 0)),
            pl.BlockSpec((D, D), lambda p, i: (0, 0)),
            pl.BlockSpec((1, D), lambda p, i: (0, 0)),
            pl.BlockSpec((D, D), lambda p, i: (0, 0)),
            pl.BlockSpec((1, D), lambda p, i: (0, 0)),
            pl.BlockSpec((1, D), lambda p, i: (0, 0)),
        ],
        out_specs=pl.BlockSpec((BR, D), lambda p, i: (i, 0)),
        out_shape=jax.ShapeDtypeStruct((N_NODES, D), jnp.float32),
        scratch_shapes=[
            pltpu.VMEM((N_NODES, D), jnp.float32),
            pltpu.VMEM((8, D), jnp.float32),
        ],
    )(summed, cnt2, x, Wl, bl.reshape(1, D), Wr,
      gamma.reshape(1, D), beta.reshape(1, D))

    return out


# speedup vs baseline: 1.0405x; 1.0082x over previous
import functools

import jax
import jax.numpy as jnp
from jax import lax
from jax.experimental import pallas as pl
from jax.experimental.pallas import tpu as pltpu
from jax.experimental.pallas import tpu_sc as plsc

N_NODES = 10000
D = 256
DQ = 64
EPS = 1e-5
N_EDGES = 160000

NT = 16
EPT = N_EDGES // NT
CHUNK = 80
NCHUNK = 128
NCHUNK_ALLOC = NCHUNK + 4
EPT_PAD = NCHUNK_ALLOC * CHUNK
N_ACC = 10240
ROWS_PER_TILE = N_ACC // NT
TSTRIPE = 624
TLAST = N_NODES - 15 * TSTRIPE


def _sc_segment_sum(src_t, dst_t, x, zeros_q, ones_c):
    mesh = plsc.VectorSubcoreMesh(core_axis_name="c", subcore_axis_name="s")

    @functools.partial(
        pl.kernel,
        mesh=mesh,
        compiler_params=pltpu.CompilerParams(use_tc_tiling_on_sc=False),
        out_type=(
            jax.ShapeDtypeStruct((4, N_ACC, DQ), jnp.float32),
            jax.ShapeDtypeStruct((N_ACC,), jnp.float32),
        ),
        scratch_types=[
            pltpu.VMEM((NCHUNK_ALLOC, CHUNK), jnp.int32),
            pltpu.VMEM((NCHUNK_ALLOC, CHUNK), jnp.int32),
            pltpu.VMEM((CHUNK, DQ), jnp.float32),
            pltpu.VMEM((CHUNK, DQ), jnp.float32),
            pltpu.VMEM((CHUNK, DQ), jnp.float32),
            pltpu.VMEM((CHUNK, DQ), jnp.float32),
            pltpu.VMEM((CHUNK,), jnp.float32),
            pltpu.VMEM_SHARED((N_NODES, DQ), jnp.float32),
            pltpu.VMEM_SHARED((N_ACC, DQ), jnp.float32),
            pltpu.VMEM_SHARED((N_ACC,), jnp.float32),
            pltpu.SemaphoreType.DMA,
            pltpu.SemaphoreType.DMA,
            pltpu.SemaphoreType.DMA,
            pltpu.SemaphoreType.DMA,
            pltpu.SemaphoreType.DMA,
            pltpu.SemaphoreType.DMA,
            pltpu.SemaphoreType.DMA,
            pltpu.SemaphoreType.DMA,
            pltpu.SemaphoreType.DMA,
        ],
    )
    def seg_sum(src_hbm, dst_hbm, x_hbm, z_hbm, o_hbm,
                sum_hbm, cnt_hbm,
                src_v, dst_v, rows0, rows1, rows2, rows3, ones_v,
                tab_sh, acc_sh, cnt_sh,
                semg0, semg1, semg2, semg3, sems0, sems1, sems2, sems3, semc):
        c = lax.axis_index("c")
        s = lax.axis_index("s")

        pltpu.sync_copy(z_hbm, rows0)
        pltpu.sync_copy(o_hbm, ones_v)
        pltpu.sync_copy(src_hbm.at[s], src_v)
        pltpu.sync_copy(dst_hbm.at[s], dst_v)

        base = s * ROWS_PER_TILE

        for q in range(2):
            qidx = 2 * c + q
            qcol = qidx * DQ

            @pl.when(s < 15)
            def _():
                pltpu.sync_copy(
                    x_hbm.at[pl.ds(s * TSTRIPE, TSTRIPE), pl.ds(qcol, DQ)],
                    tab_sh.at[pl.ds(s * TSTRIPE, TSTRIPE)])

            @pl.when(s == 15)
            def _():
                pltpu.sync_copy(
                    x_hbm.at[pl.ds(15 * TSTRIPE, TLAST), pl.ds(qcol, DQ)],
                    tab_sh.at[pl.ds(15 * TSTRIPE, TLAST)])
            for k in range(ROWS_PER_TILE // CHUNK):
                pltpu.sync_copy(rows0, acc_sh.at[pl.ds(base + k * CHUNK, CHUNK)])
            if q == 0:
                for k in range(ROWS_PER_TILE // DQ):
                    pltpu.sync_copy(rows0.at[0],
                                    cnt_sh.at[pl.ds(base + k * DQ, DQ)])
            plsc.subcore_barrier()

            bufs = (rows0, rows1, rows2, rows3)
            semg = (semg0, semg1, semg2, semg3)
            sems = (sems0, sems1, sems2, sems3)

            def edge_loop(with_cnt):
                def step(j, b, warm):
                    pltpu.make_async_copy(
                        tab_sh.at[src_v.at[j]], bufs[b], semg[b]).wait()
                    if with_cnt:
                        pltpu.async_copy(ones_v, cnt_sh.at[dst_v.at[j]], semc,
                                         add=True)
                        if warm:
                            pltpu.make_async_copy(
                                ones_v, cnt_sh.at[dst_v.at[j - 2]], semc).wait()
                    pltpu.async_copy(bufs[b], acc_sh.at[dst_v.at[j]], sems[b],
                                     add=True)
                    b2 = (b + 2) % 4
                    if warm:
                        pltpu.make_async_copy(
                            bufs[b2], acc_sh.at[dst_v.at[j - 2]], sems[b2]).wait()
                    pltpu.async_copy(tab_sh.at[src_v.at[j + 2]], bufs[b2],
                                     semg[b2])

                pltpu.async_copy(tab_sh.at[src_v.at[0]], rows0, semg0)
                pltpu.async_copy(tab_sh.at[src_v.at[1]], rows1, semg1)
                step(0, 0, False)
                step(1, 1, False)

                def quad(i, carry):
                    j0 = 4 * i + 2
                    step(j0, 2, True)
                    step(j0 + 1, 3, True)
                    step(j0 + 2, 0, True)
                    step(j0 + 3, 1, True)
                    return carry

                lax.fori_loop(0, (NCHUNK_ALLOC - 4) // 4, quad, 0)
                nl = NCHUNK_ALLOC
                if with_cnt:
                    pltpu.make_async_copy(
                        ones_v, cnt_sh.at[dst_v.at[nl - 4]], semc).wait()
                    pltpu.make_async_copy(
                        ones_v, cnt_sh.at[dst_v.at[nl - 3]], semc).wait()
                pltpu.make_async_copy(
                    rows0, acc_sh.at[dst_v.at[nl - 4]], sems0).wait()
                pltpu.make_async_copy(
                    rows1, acc_sh.at[dst_v.at[nl - 3]], sems1).wait()
                pltpu.make_async_copy(
                    tab_sh.at[src_v.at[nl - 2]], rows2, semg2).wait()
                pltpu.make_async_copy(
                    tab_sh.at[src_v.at[nl - 1]], rows3, semg3).wait()

            if q == 0:
                @pl.when(c == 0)
                def _():
                    edge_loop(True)

                @pl.when(c == 1)
                def _():
                    edge_loop(False)
            else:
                edge_loop(False)

            plsc.subcore_barrier()

            pltpu.sync_copy(
                acc_sh.at[pl.ds(base, ROWS_PER_TILE)],
                sum_hbm.at[qidx, pl.ds(base, ROWS_PER_TILE)])
            if q == 0:
                @pl.when(c == 0)
                def _():
                    pltpu.sync_copy(cnt_sh.at[pl.ds(base, ROWS_PER_TILE)],
                                    cnt_hbm.at[pl.ds(base, ROWS_PER_TILE)])
            if q == 0:
                pltpu.sync_copy(z_hbm, rows0)

    return seg_sum(src_t, dst_t, x, zeros_q, ones_c)


BR = 1000
NB = N_NODES // BR


def _tc_fused_body(sum_ref, cnt_ref, x_ref, wl_ref, bl_ref, wr_ref,
                   gamma_ref, beta_ref, out_ref, conv_scr, acc_ref):
    p = pl.program_id(0)
    i = pl.program_id(1)

    @pl.when(p == 0)
    def _():
        recip = 1.0 / jnp.maximum(cnt_ref[...], 1.0)
        dn = (((1,), (1,)), ((), ()))
        wl = wl_ref[...]
        conv = lax.dot_general(x_ref[...], wr_ref[...], dn,
                               preferred_element_type=jnp.float32)
        for qq in range(4):
            mean_q = sum_ref[qq] * recip
            conv += lax.dot_general(mean_q, wl[:, qq * DQ:(qq + 1) * DQ], dn,
                                    preferred_element_type=jnp.float32)
        conv += bl_ref[...]
        conv_scr[pl.ds(i * BR, BR), :] = conv

        @pl.when(i == 0)
        def _():
            acc_ref[...] = jnp.zeros_like(acc_ref)

        acc_ref[0:1, :] += jnp.sum(conv, axis=0, keepdims=True)
        acc_ref[1:2, :] += jnp.sum(conv * conv, axis=0, keepdims=True)

    @pl.when(p == 1)
    def _():
        n = jnp.float32(N_NODES)
        mu = acc_ref[0:1, :] / n
        var = acc_ref[1:2, :] / n - mu * mu
        inv = lax.rsqrt(var + EPS)
        conv = conv_scr[pl.ds(i * BR, BR), :]
        bn = gamma_ref[...] * (conv - mu) * inv + beta_ref[...]
        out_ref[...] = jnp.maximum(bn, 0.0) + x_ref[...]


def kernel(x, ei, Wl, bl, Wr, gamma, beta):
    src = ei[0].astype(jnp.int32)
    dst = ei[1].astype(jnp.int32)

    pad = EPT_PAD - EPT
    src_t = jnp.concatenate(
        [src.reshape(NT, EPT), jnp.zeros((NT, pad), jnp.int32)], axis=1
    ).reshape(NT, NCHUNK_ALLOC, CHUNK)
    dst_t = jnp.concatenate(
        [dst.reshape(NT, EPT), jnp.full((NT, pad), N_NODES, jnp.int32)], axis=1
    ).reshape(NT, NCHUNK_ALLOC, CHUNK)

    zeros_q = jnp.zeros((CHUNK, DQ), jnp.float32)
    ones_c = jnp.ones((CHUNK,), jnp.float32)

    summed, cnt = _sc_segment_sum(src_t, dst_t, x, zeros_q, ones_c)
    cnt2 = cnt.reshape(N_ACC, 1)

    out = pl.pallas_call(
        _tc_fused_body,
        grid=(2, NB),
        in_specs=[
            pl.BlockSpec((4, BR, DQ), lambda p, i: (0, (1 - p) * i, 0)),
            pl.BlockSpec((BR, 1), lambda p, i: ((1 - p) * i, 0)),
            pl.BlockSpec((BR, D), lambda p, i: (i, 0)),
            pl.BlockSpec((D, D), lambda p, i: (0, 0)),
            pl.BlockSpec((1, D), lambda p, i: (0, 0)),
            pl.BlockSpec((D, D), lambda p, i: (0, 0)),
            pl.BlockSpec((1, D), lambda p, i: (0, 0)),
            pl.BlockSpec((1, D), lambda p, i: (0, 0)),
        ],
        out_specs=pl.BlockSpec((BR, D), lambda p, i: (i, 0)),
        out_shape=jax.ShapeDtypeStruct((N_NODES, D), jnp.float32),
        scratch_shapes=[
            pltpu.VMEM((N_NODES, D), jnp.float32),
            pltpu.VMEM((8, D), jnp.float32),
        ],
    )(summed, cnt2, x, Wl, bl.reshape(1, D), Wr,
      gamma.reshape(1, D), beta.reshape(1, D))

    return out
